# Initial kernel scaffold; baseline (speedup 1.0000x reference)
import jax
import jax.numpy as jnp
from jax.experimental import pallas as pl


def kernel(x, edge_index, batch, W1, b1, W2, b2, Wfc, bfc):
    raise NotImplementedError("write your pallas kernel here")


# speedup vs baseline: 3.1671x; 3.1671x over previous
import functools

import jax
import jax.numpy as jnp
from jax import lax
from jax.experimental import pallas as pl
from jax.experimental.pallas import tpu as pltpu

N_NODES = 10000
N_GRAPHS = 64
ROW_BLK = 1000


def _pre_body(deg_ref, x_ref, dinv_ref, xs_ref):
    dinv = lax.rsqrt(deg_ref[...])
    dinv_ref[...] = dinv
    xs_ref[...] = dinv * x_ref[...]


def _layer1_body(agg_ref, xs_ref, dinv_ref, w_ref, b_ref, o_ref):
    u = dinv_ref[...] * (agg_ref[...] + xs_ref[...])
    h = jnp.dot(u, w_ref[...], preferred_element_type=jnp.float32) + b_ref[...]
    o_ref[...] = dinv_ref[...] * jnp.maximum(h, 0.0)


def _layer2_body(agg_ref, hs_ref, dinv_ref, w_ref, b_ref, wfc_ref, o_ref):
    u = dinv_ref[...] * (agg_ref[...] + hs_ref[...])
    h = jnp.dot(u, w_ref[...], preferred_element_type=jnp.float32) + b_ref[...]
    h = jnp.maximum(h, 0.0)
    o_ref[...] = jnp.dot(h, wfc_ref[...], preferred_element_type=jnp.float32)


def _pool_body(z_ref, bt_ref, bfc_ref, o_ref):
    z = z_ref[...]
    bt = bt_ref[...]
    gids = lax.broadcasted_iota(jnp.int32, (N_GRAPHS, z.shape[0], z.shape[1]), 0)
    m = bt[None, :, :] == gids
    s1 = jnp.sum(jnp.where(m, z[None, :, :], 0.0), axis=1)
    sums = jnp.sum(s1, axis=1, keepdims=True)
    c1 = jnp.sum(jnp.where(m, 1.0, 0.0), axis=1)
    cnts = jnp.sum(c1, axis=1, keepdims=True)
    o_ref[...] = sums / jnp.maximum(cnts, 1.0) + bfc_ref[...]


def _row_spec(width):
    return pl.BlockSpec((ROW_BLK, width), lambda i: (i, 0))


def _full_spec(shape):
    return pl.BlockSpec(shape, lambda i: tuple(0 for _ in shape))


def _tc_pre(deg, x):
    return pl.pallas_call(
        _pre_body,
        grid=(N_NODES // ROW_BLK,),
        in_specs=[_row_spec(1), _row_spec(x.shape[1])],
        out_specs=[_row_spec(1), _row_spec(x.shape[1])],
        out_shape=[
            jax.ShapeDtypeStruct((N_NODES, 1), jnp.float32),
            jax.ShapeDtypeStruct((N_NODES, x.shape[1]), jnp.float32),
        ],
    )(deg, x)


def _tc_layer1(agg, xs, dinv, W1, b1):
    f_in, f_out = W1.shape
    return pl.pallas_call(
        _layer1_body,
        grid=(N_NODES // ROW_BLK,),
        in_specs=[
            _row_spec(f_in), _row_spec(f_in), _row_spec(1),
            _full_spec((f_in, f_out)), _full_spec((1, f_out)),
        ],
        out_specs=_row_spec(f_out),
        out_shape=jax.ShapeDtypeStruct((N_NODES, f_out), jnp.float32),
    )(agg, xs, dinv, W1, b1.reshape(1, f_out))


def _tc_layer2(agg, hs, dinv, W2, b2, Wfc):
    f_in, f_out = W2.shape
    return pl.pallas_call(
        _layer2_body,
        grid=(N_NODES // ROW_BLK,),
        in_specs=[
            _row_spec(f_in), _row_spec(f_in), _row_spec(1),
            _full_spec((f_in, f_out)), _full_spec((1, f_out)),
            _full_spec((f_out, 1)),
        ],
        out_specs=_row_spec(1),
        out_shape=jax.ShapeDtypeStruct((N_NODES, 1), jnp.float32),
    )(agg, hs, dinv, W2, b2.reshape(1, f_out), Wfc)


def _tc_pool(z, batch_i32, bfc):
    zp = jnp.concatenate([z[:, 0], jnp.zeros((240,), jnp.float32)]).reshape(80, 128)
    bp = jnp.concatenate(
        [batch_i32, jnp.full((240,), 1 << 20, jnp.int32)]).reshape(80, 128)
    return pl.pallas_call(
        _pool_body,
        in_specs=[
            pl.BlockSpec((80, 128), lambda: (0, 0)),
            pl.BlockSpec((80, 128), lambda: (0, 0)),
            pl.BlockSpec((1, 1), lambda: (0, 0)),
        ],
        out_specs=pl.BlockSpec((N_GRAPHS, 1), lambda: (0, 0)),
        out_shape=jax.ShapeDtypeStruct((N_GRAPHS, 1), jnp.float32),
    )(zp, bp, bfc.reshape(1, 1))


def _aggregate(rows, src, dst):
    z = jnp.zeros((N_NODES, rows.shape[1]), jnp.float32)
    return z.at[dst].add(rows[src])


def _degrees(dst):
    z = jnp.zeros((N_NODES,), jnp.float32)
    return z.at[dst].add(1.0)


def kernel(x, edge_index, batch, W1, b1, W2, b2, Wfc, bfc):
    src = edge_index[0].astype(jnp.int32)
    dst = edge_index[1].astype(jnp.int32)
    batch_i32 = batch.astype(jnp.int32)

    deg = (_degrees(dst) + 1.0).reshape(N_NODES, 1)
    dinv, xs = _tc_pre(deg, x)

    agg1 = _aggregate(xs, src, dst)
    hs1 = _tc_layer1(agg1, xs, dinv, W1, b1)

    agg2 = _aggregate(hs1, src, dst)
    z = _tc_layer2(agg2, hs1, dinv, W2, b2, Wfc)

    return _tc_pool(z, batch_i32, bfc)


# speedup vs baseline: 3.7513x; 1.1845x over previous
import functools

import jax
import jax.numpy as jnp
from jax import lax
from jax.experimental import pallas as pl
from jax.experimental.pallas import tpu as pltpu
from jax.experimental.pallas import tpu_sc as plsc

N_NODES = 10000
N_GRAPHS = 64
ROW_BLK = 1000
CH = 128

NSUB = 16
CHUNK = 128
CHUNKS_PER_SUB = 160
E_PAD = NSUB * CHUNKS_PER_SUB * CHUNK
NCHUNK_ROWS = E_PAD // CHUNK
HALF_R = 5120
ACC_R = 5248
JUNK = HALF_R
NR_OUT = 2 * HALF_R
ZERO_PER_SUB = HALF_R // NSUB
DEG_R = 10112
DEG_PER_SUB = DEG_R // NSUB

@functools.lru_cache(maxsize=None)
def _sc_mesh():
    return plsc.VectorSubcoreMesh(core_axis_name="c", subcore_axis_name="s")


def _deg_body(dst0, dst1, ones, zeros, out_l, out_r, idx_d0, idx_d1, ones_v,
              acc):
    cid = lax.axis_index("c")
    sid = lax.axis_index("s")
    half = CHUNKS_PER_SUB // 2
    base = cid * (NCHUNK_ROWS // 2) + sid * half
    pltpu.sync_copy(dst0.at[pl.ds(base, half)], idx_d0)
    pltpu.sync_copy(dst1.at[pl.ds(base, half)], idx_d1)
    pltpu.sync_copy(ones, ones_v)

    for p, idx_d in ((0, idx_d0), (1, idx_d1)):
        pltpu.sync_copy(zeros.at[pl.ds(sid * ZERO_PER_SUB, ZERO_PER_SUB)],
                        acc.at[pl.ds(sid * ZERO_PER_SUB, ZERO_PER_SUB)])
        plsc.subcore_barrier()

        def step(j, carry):
            pltpu.sync_copy(ones_v, acc.at[idx_d.at[j]], add=True)
            return carry

        lax.fori_loop(0, half, step, 0)
        plsc.subcore_barrier()

        obase = p * HALF_R + sid * ZERO_PER_SUB

        @pl.when(cid == 0)
        def _():
            pltpu.sync_copy(acc.at[pl.ds(sid * ZERO_PER_SUB, ZERO_PER_SUB)],
                            out_l.at[pl.ds(obase, ZERO_PER_SUB)])

        @pl.when(cid == 1)
        def _():
            pltpu.sync_copy(acc.at[pl.ds(sid * ZERO_PER_SUB, ZERO_PER_SUB)],
                            out_r.at[pl.ds(obase, ZERO_PER_SUB)])
        plsc.subcore_barrier()


def _sc_degrees(dst0, dst1):
    deg_kernel = pl.kernel(
        _deg_body,
        out_type=[
            jax.ShapeDtypeStruct((NR_OUT, CH), jnp.float32),
            jax.ShapeDtypeStruct((NR_OUT, CH), jnp.float32),
        ],
        mesh=_sc_mesh(),
        scratch_types=[
            pltpu.VMEM((CHUNKS_PER_SUB // 2, CHUNK), jnp.int32),
            pltpu.VMEM((CHUNKS_PER_SUB // 2, CHUNK), jnp.int32),
            pltpu.VMEM((CHUNK, CH), jnp.float32),
            pltpu.VMEM_SHARED((ACC_R, CH), jnp.float32),
        ],
    )
    ones = jnp.ones((CHUNK, CH), jnp.float32)
    zeros = jnp.zeros((HALF_R, CH), jnp.float32)
    return deg_kernel(dst0, dst1, ones, zeros)


def _agg_body(table, srcl, srcr, dst0, dst1, zeros, out_l, out_r,
              idx_s, idx_d0, idx_d1, rows, acc, sem):
    cid = lax.axis_index("c")
    sid = lax.axis_index("s")
    base = sid * CHUNKS_PER_SUB

    @pl.when(cid == 0)
    def _():
        pltpu.sync_copy(srcl.at[pl.ds(base, CHUNKS_PER_SUB)], idx_s)

    @pl.when(cid == 1)
    def _():
        pltpu.sync_copy(srcr.at[pl.ds(base, CHUNKS_PER_SUB)], idx_s)

    pltpu.sync_copy(dst0.at[pl.ds(base, CHUNKS_PER_SUB)], idx_d0)
    pltpu.sync_copy(dst1.at[pl.ds(base, CHUNKS_PER_SUB)], idx_d1)

    for p, idx_d in ((0, idx_d0), (1, idx_d1)):
        pltpu.sync_copy(zeros.at[pl.ds(sid * ZERO_PER_SUB, ZERO_PER_SUB)],
                        acc.at[pl.ds(sid * ZERO_PER_SUB, ZERO_PER_SUB)])
        plsc.subcore_barrier()

        def step(j, carry):
            pltpu.async_copy(table.at[idx_s.at[j]], rows, sem).wait()
            pltpu.sync_copy(rows, acc.at[idx_d.at[j]], add=True)
            return carry

        lax.fori_loop(0, CHUNKS_PER_SUB, step, 0)
        plsc.subcore_barrier()

        obase = p * HALF_R + sid * ZERO_PER_SUB

        @pl.when(cid == 0)
        def _():
            pltpu.sync_copy(acc.at[pl.ds(sid * ZERO_PER_SUB, ZERO_PER_SUB)],
                            out_l.at[pl.ds(obase, ZERO_PER_SUB)])

        @pl.when(cid == 1)
        def _():
            pltpu.sync_copy(acc.at[pl.ds(sid * ZERO_PER_SUB, ZERO_PER_SUB)],
                            out_r.at[pl.ds(obase, ZERO_PER_SUB)])
        plsc.subcore_barrier()


@functools.lru_cache(maxsize=None)
def _make_agg_kernel():
    return pl.kernel(
        _agg_body,
        out_type=[
            jax.ShapeDtypeStruct((NR_OUT, CH), jnp.float32),
            jax.ShapeDtypeStruct((NR_OUT, CH), jnp.float32),
        ],
        mesh=_sc_mesh(),
        scratch_types=[
            pltpu.VMEM((CHUNKS_PER_SUB, CHUNK), jnp.int32),
            pltpu.VMEM((CHUNKS_PER_SUB, CHUNK), jnp.int32),
            pltpu.VMEM((CHUNKS_PER_SUB, CHUNK), jnp.int32),
            pltpu.VMEM((CHUNK, CH), jnp.float32),
            pltpu.VMEM_SHARED((ACC_R, CH), jnp.float32),
            pltpu.SemaphoreType.DMA,
        ],
    )


def _sc_aggregate(rows_mat, srcl, srcr, dst0, dst1):
    table = jnp.concatenate(
        [rows_mat, jnp.zeros((1, 2 * CH), jnp.float32)]).reshape(-1, CH)
    zeros = jnp.zeros((HALF_R, CH), jnp.float32)
    return _make_agg_kernel()(table, srcl, srcr, dst0, dst1, zeros)


def _pre_body(d0_ref, d1_ref, corr_ref, x_ref, w_ref, dinv_ref, p_ref):
    deg = d0_ref[...] + d1_ref[...] + 1.0 - corr_ref[...]
    dinv = lax.rsqrt(deg)
    dinv_ref[...] = dinv
    p_ref[...] = jnp.dot(dinv * x_ref[...], w_ref[...],
                         preferred_element_type=jnp.float32)


def _mid_body(aggl_ref, aggr_ref, p_ref, dinv_ref, b_ref, w_ref, p2_ref):
    agg = jnp.concatenate([aggl_ref[...], aggr_ref[...]], axis=1)
    h = dinv_ref[...] * (agg + p_ref[...]) + b_ref[...]
    hs = dinv_ref[...] * jnp.maximum(h, 0.0)
    p2_ref[...] = jnp.dot(hs, w_ref[...], preferred_element_type=jnp.float32)


def _fin_body(aggl_ref, aggr_ref, p_ref, dinv_ref, b_ref, wfc_ref, z_ref):
    agg = jnp.concatenate([aggl_ref[...], aggr_ref[...]], axis=1)
    h = dinv_ref[...] * (agg + p_ref[...]) + b_ref[...]
    h = jnp.maximum(h, 0.0)
    z_ref[...] = jnp.dot(h, wfc_ref[...], preferred_element_type=jnp.float32)


def _pool_body(z_ref, bt_ref, bfc_ref, o_ref):
    z = z_ref[...]
    bt = bt_ref[...]
    gids = lax.broadcasted_iota(jnp.int32, (N_GRAPHS, z.shape[0], z.shape[1]), 0)
    m = bt[None, :, :] == gids
    s1 = jnp.sum(jnp.where(m, z[None, :, :], 0.0), axis=1)
    sums = jnp.sum(s1, axis=1, keepdims=True)
    c1 = jnp.sum(jnp.where(m, 1.0, 0.0), axis=1)
    cnts = jnp.sum(c1, axis=1, keepdims=True)
    o_ref[...] = sums / jnp.maximum(cnts, 1.0) + bfc_ref[...]


def _row_spec(width):
    return pl.BlockSpec((ROW_BLK, width), lambda i: (i, 0))


def _full_spec(shape):
    return pl.BlockSpec(shape, lambda i: tuple(0 for _ in shape))


def _tc_pre(d0, d1, corr, x, W1):
    f_in, f_out = W1.shape
    return pl.pallas_call(
        _pre_body,
        grid=(N_NODES // ROW_BLK,),
        in_specs=[_row_spec(1), _row_spec(1), _row_spec(1), _row_spec(f_in),
                  _full_spec((f_in, f_out))],
        out_specs=[_row_spec(1), _row_spec(f_out)],
        out_shape=[
            jax.ShapeDtypeStruct((N_NODES, 1), jnp.float32),
            jax.ShapeDtypeStruct((N_NODES, f_out), jnp.float32),
        ],
    )(d0, d1, corr, x, W1)


def _tc_mid(aggl, aggr, p, dinv, b1, W2):
    f_in, f_out = W2.shape
    return pl.pallas_call(
        _mid_body,
        grid=(N_NODES // ROW_BLK,),
        in_specs=[
            _row_spec(CH), _row_spec(CH), _row_spec(f_in), _row_spec(1),
            _full_spec((1, f_in)), _full_spec((f_in, f_out)),
        ],
        out_specs=_row_spec(f_out),
        out_shape=jax.ShapeDtypeStruct((N_NODES, f_out), jnp.float32),
    )(aggl, aggr, p, dinv, b1.reshape(1, f_in), W2)


def _tc_fin(aggl, aggr, p, dinv, b2, Wfc):
    f_in = Wfc.shape[0]
    return pl.pallas_call(
        _fin_body,
        grid=(N_NODES // ROW_BLK,),
        in_specs=[
            _row_spec(CH), _row_spec(CH), _row_spec(f_in), _row_spec(1),
            _full_spec((1, f_in)), _full_spec((f_in, 1)),
        ],
        out_specs=_row_spec(1),
        out_shape=jax.ShapeDtypeStruct((N_NODES, 1), jnp.float32),
    )(aggl, aggr, p, dinv, b2.reshape(1, f_in), Wfc)


def _tc_pool(z, batch_i32, bfc):
    zp = jnp.concatenate([z[:, 0], jnp.zeros((240,), jnp.float32)]).reshape(80, 128)
    bp = jnp.concatenate(
        [batch_i32, jnp.full((240,), 1 << 20, jnp.int32)]).reshape(80, 128)
    return pl.pallas_call(
        _pool_body,
        in_specs=[
            pl.BlockSpec((80, 128), lambda: (0, 0)),
            pl.BlockSpec((80, 128), lambda: (0, 0)),
            pl.BlockSpec((1, 1), lambda: (0, 0)),
        ],
        out_specs=pl.BlockSpec((N_GRAPHS, 1), lambda: (0, 0)),
        out_shape=jax.ShapeDtypeStruct((N_GRAPHS, 1), jnp.float32),
    )(zp, bp, bfc.reshape(1, 1))


def kernel(x, edge_index, batch, W1, b1, W2, b2, Wfc, bfc):
    src = edge_index[0].astype(jnp.int32)
    dst = edge_index[1].astype(jnp.int32)
    batch_i32 = batch.astype(jnp.int32)

    n_pad = E_PAD - src.shape[0]
    src_pad = jnp.concatenate([src, jnp.full((n_pad,), N_NODES, jnp.int32)])
    dst_pad = jnp.concatenate([dst, jnp.zeros((n_pad,), jnp.int32)])
    srcl = (2 * src_pad).reshape(NCHUNK_ROWS, CHUNK)
    srcr = (2 * src_pad + 1).reshape(NCHUNK_ROWS, CHUNK)
    dstc = dst_pad.reshape(NCHUNK_ROWS, CHUNK)
    dst0 = jnp.where(dst_pad < HALF_R, dst_pad, JUNK).reshape(NCHUNK_ROWS, CHUNK)
    dst1 = jnp.where(dst_pad >= HALF_R, dst_pad - HALF_R,
                     JUNK).reshape(NCHUNK_ROWS, CHUNK)

    d0, d1 = _sc_degrees(dst0, dst1)
    corr = jnp.zeros((N_NODES, 1), jnp.float32).at[0, 0].set(float(n_pad))
    dinv, p1 = _tc_pre(d0[:N_NODES, :1], d1[:N_NODES, :1], corr, x, W1)

    a1l, a1r = _sc_aggregate(p1, srcl, srcr, dst0, dst1)
    p2 = _tc_mid(a1l[:N_NODES], a1r[:N_NODES], p1, dinv, b1, W2)

    a2l, a2r = _sc_aggregate(p2, srcl, srcr, dst0, dst1)
    z = _tc_fin(a2l[:N_NODES], a2r[:N_NODES], p2, dinv, b2, Wfc)

    return _tc_pool(z, batch_i32, bfc)


# speedup vs baseline: 4.2884x; 1.1432x over previous
import functools

import jax
import jax.numpy as jnp
from jax import lax
from jax.experimental import pallas as pl
from jax.experimental.pallas import tpu as pltpu
from jax.experimental.pallas import tpu_sc as plsc

N_NODES = 10000
N_GRAPHS = 64
ROW_BLK = 1000
CH = 128

NSUB = 16
CHUNK = 128
CHUNKS_PER_SUB = 160
E_PAD = NSUB * CHUNKS_PER_SUB * CHUNK
NCHUNK_ROWS = E_PAD // CHUNK
HALF_R = 5120
ACC_R = 5248
JUNK = HALF_R
NR_OUT = 2 * HALF_R
ZERO_PER_SUB = HALF_R // NSUB
DEG_R = 10112
DEG_PER_SUB = DEG_R // NSUB

@functools.lru_cache(maxsize=None)
def _sc_mesh():
    return plsc.VectorSubcoreMesh(core_axis_name="c", subcore_axis_name="s")


def _deg_body(dst0, dst1, ones, zeros, out_l, out_r, idx_d0, idx_d1, ones_v,
              acc):
    cid = lax.axis_index("c")
    sid = lax.axis_index("s")
    half = CHUNKS_PER_SUB // 2
    base = cid * (NCHUNK_ROWS // 2) + sid * half
    pltpu.sync_copy(dst0.at[pl.ds(base, half)], idx_d0)
    pltpu.sync_copy(dst1.at[pl.ds(base, half)], idx_d1)
    pltpu.sync_copy(ones, ones_v)

    for p, idx_d in ((0, idx_d0), (1, idx_d1)):
        pltpu.sync_copy(zeros.at[pl.ds(sid * ZERO_PER_SUB, ZERO_PER_SUB)],
                        acc.at[pl.ds(sid * ZERO_PER_SUB, ZERO_PER_SUB)])
        plsc.subcore_barrier()

        def step(j, carry):
            pltpu.sync_copy(ones_v, acc.at[idx_d.at[j]], add=True)
            return carry

        lax.fori_loop(0, half, step, 0)
        plsc.subcore_barrier()

        obase = p * HALF_R + sid * ZERO_PER_SUB

        @pl.when(cid == 0)
        def _():
            pltpu.sync_copy(acc.at[pl.ds(sid * ZERO_PER_SUB, ZERO_PER_SUB)],
                            out_l.at[pl.ds(obase, ZERO_PER_SUB)])

        @pl.when(cid == 1)
        def _():
            pltpu.sync_copy(acc.at[pl.ds(sid * ZERO_PER_SUB, ZERO_PER_SUB)],
                            out_r.at[pl.ds(obase, ZERO_PER_SUB)])
        plsc.subcore_barrier()


def _sc_degrees(dst0, dst1):
    deg_kernel = pl.kernel(
        _deg_body,
        out_type=[
            jax.ShapeDtypeStruct((NR_OUT, CH), jnp.float32),
            jax.ShapeDtypeStruct((NR_OUT, CH), jnp.float32),
        ],
        mesh=_sc_mesh(),
        scratch_types=[
            pltpu.VMEM((CHUNKS_PER_SUB // 2, CHUNK), jnp.int32),
            pltpu.VMEM((CHUNKS_PER_SUB // 2, CHUNK), jnp.int32),
            pltpu.VMEM((CHUNK, CH), jnp.float32),
            pltpu.VMEM_SHARED((ACC_R, CH), jnp.float32),
        ],
    )
    ones = jnp.ones((CHUNK, CH), jnp.float32)
    zeros = jnp.zeros((HALF_R, CH), jnp.float32)
    return deg_kernel(dst0, dst1, ones, zeros)


NBUF = 2


def _agg_body(table, srcl, srcr, dst0, dst1, zeros, out_l, out_r,
              idx_s, idx_d, b0, b1, acc, g0, g1):
    cid = lax.axis_index("c")
    sid = lax.axis_index("s")
    base = sid * CHUNKS_PER_SUB
    bufs = (b0, b1)
    gsem = (g0, g1)
    ngroups = CHUNKS_PER_SUB // NBUF

    @pl.when(cid == 0)
    def _():
        pltpu.sync_copy(srcl.at[pl.ds(base, CHUNKS_PER_SUB)], idx_s)

    @pl.when(cid == 1)
    def _():
        pltpu.sync_copy(srcr.at[pl.ds(base, CHUNKS_PER_SUB)], idx_s)

    for p, dstp in ((0, dst0), (1, dst1)):
        pltpu.sync_copy(dstp.at[pl.ds(base, CHUNKS_PER_SUB)], idx_d)
        pltpu.sync_copy(zeros.at[pl.ds(sid * ZERO_PER_SUB, ZERO_PER_SUB)],
                        acc.at[pl.ds(sid * ZERO_PER_SUB, ZERO_PER_SUB)])
        plsc.subcore_barrier()

        for b in range(NBUF):
            pltpu.async_copy(table.at[idx_s.at[b]], bufs[b], gsem[b])

        def group(g, carry):
            jb = g * NBUF
            for b in range(NBUF):
                j = jb + b
                pltpu.make_async_copy(table.at[idx_s.at[j]], bufs[b],
                                      gsem[b]).wait()
                pltpu.sync_copy(bufs[b], acc.at[idx_d.at[j]], add=True)
                pltpu.async_copy(table.at[idx_s.at[j + NBUF]], bufs[b],
                                 gsem[b])
            return carry

        lax.fori_loop(0, ngroups - 1, group, 0)

        for b in range(NBUF):
            j = CHUNKS_PER_SUB - NBUF + b
            pltpu.make_async_copy(table.at[idx_s.at[j]], bufs[b],
                                  gsem[b]).wait()
            pltpu.sync_copy(bufs[b], acc.at[idx_d.at[j]], add=True)

        plsc.subcore_barrier()
        obase = p * HALF_R + sid * ZERO_PER_SUB

        @pl.when(cid == 0)
        def _():
            pltpu.sync_copy(acc.at[pl.ds(sid * ZERO_PER_SUB, ZERO_PER_SUB)],
                            out_l.at[pl.ds(obase, ZERO_PER_SUB)])

        @pl.when(cid == 1)
        def _():
            pltpu.sync_copy(acc.at[pl.ds(sid * ZERO_PER_SUB, ZERO_PER_SUB)],
                            out_r.at[pl.ds(obase, ZERO_PER_SUB)])
        plsc.subcore_barrier()


@functools.lru_cache(maxsize=None)
def _make_agg_kernel():
    return pl.kernel(
        _agg_body,
        out_type=[
            jax.ShapeDtypeStruct((NR_OUT, CH), jnp.float32),
            jax.ShapeDtypeStruct((NR_OUT, CH), jnp.float32),
        ],
        mesh=_sc_mesh(),
        scratch_types=[
            pltpu.VMEM((CHUNKS_PER_SUB, CHUNK), jnp.int32),
            pltpu.VMEM((CHUNKS_PER_SUB, CHUNK), jnp.int32),
            pltpu.VMEM((CHUNK, CH), jnp.float32),
            pltpu.VMEM((CHUNK, CH), jnp.float32),
            pltpu.VMEM_SHARED((ACC_R, CH), jnp.float32),
            pltpu.SemaphoreType.DMA,
            pltpu.SemaphoreType.DMA,
        ],
    )


def _sc_aggregate(rows_mat, srcl, srcr, dst0, dst1):
    table = jnp.concatenate(
        [rows_mat, jnp.zeros((1, 2 * CH), jnp.float32)]).reshape(-1, CH)
    zeros = jnp.zeros((HALF_R, CH), jnp.float32)
    return _make_agg_kernel()(table, srcl, srcr, dst0, dst1, zeros)


def _pre_body(d0_ref, d1_ref, corr_ref, x_ref, w_ref, dinv_ref, p_ref):
    deg = d0_ref[...] + d1_ref[...] + 1.0 - corr_ref[...]
    dinv = lax.rsqrt(deg)
    dinv_ref[...] = dinv
    p_ref[...] = jnp.dot(dinv * x_ref[...], w_ref[...],
                         preferred_element_type=jnp.float32)


def _mid_body(aggl_ref, aggr_ref, p_ref, dinv_ref, b_ref, w_ref, p2_ref):
    agg = jnp.concatenate([aggl_ref[...], aggr_ref[...]], axis=1)
    h = dinv_ref[...] * (agg + p_ref[...]) + b_ref[...]
    hs = dinv_ref[...] * jnp.maximum(h, 0.0)
    p2_ref[...] = jnp.dot(hs, w_ref[...], preferred_element_type=jnp.float32)


def _fin_body(aggl_ref, aggr_ref, p_ref, dinv_ref, b_ref, wfc_ref, z_ref):
    agg = jnp.concatenate([aggl_ref[...], aggr_ref[...]], axis=1)
    h = dinv_ref[...] * (agg + p_ref[...]) + b_ref[...]
    h = jnp.maximum(h, 0.0)
    z_ref[...] = jnp.dot(h, wfc_ref[...], preferred_element_type=jnp.float32)


def _pool_body(z_ref, bt_ref, bfc_ref, o_ref):
    z = z_ref[...]
    bt = bt_ref[...]
    gids = lax.broadcasted_iota(jnp.int32, (N_GRAPHS, z.shape[0], z.shape[1]), 0)
    m = bt[None, :, :] == gids
    s1 = jnp.sum(jnp.where(m, z[None, :, :], 0.0), axis=1)
    sums = jnp.sum(s1, axis=1, keepdims=True)
    c1 = jnp.sum(jnp.where(m, 1.0, 0.0), axis=1)
    cnts = jnp.sum(c1, axis=1, keepdims=True)
    o_ref[...] = sums / jnp.maximum(cnts, 1.0) + bfc_ref[...]


def _row_spec(width):
    return pl.BlockSpec((ROW_BLK, width), lambda i: (i, 0))


def _full_spec(shape):
    return pl.BlockSpec(shape, lambda i: tuple(0 for _ in shape))


def _tc_pre(d0, d1, corr, x, W1):
    f_in, f_out = W1.shape
    return pl.pallas_call(
        _pre_body,
        grid=(N_NODES // ROW_BLK,),
        in_specs=[_row_spec(1), _row_spec(1), _row_spec(1), _row_spec(f_in),
                  _full_spec((f_in, f_out))],
        out_specs=[_row_spec(1), _row_spec(f_out)],
        out_shape=[
            jax.ShapeDtypeStruct((N_NODES, 1), jnp.float32),
            jax.ShapeDtypeStruct((N_NODES, f_out), jnp.float32),
        ],
    )(d0, d1, corr, x, W1)


def _tc_mid(aggl, aggr, p, dinv, b1, W2):
    f_in, f_out = W2.shape
    return pl.pallas_call(
        _mid_body,
        grid=(N_NODES // ROW_BLK,),
        in_specs=[
            _row_spec(CH), _row_spec(CH), _row_spec(f_in), _row_spec(1),
            _full_spec((1, f_in)), _full_spec((f_in, f_out)),
        ],
        out_specs=_row_spec(f_out),
        out_shape=jax.ShapeDtypeStruct((N_NODES, f_out), jnp.float32),
    )(aggl, aggr, p, dinv, b1.reshape(1, f_in), W2)


def _tc_fin(aggl, aggr, p, dinv, b2, Wfc):
    f_in = Wfc.shape[0]
    return pl.pallas_call(
        _fin_body,
        grid=(N_NODES // ROW_BLK,),
        in_specs=[
            _row_spec(CH), _row_spec(CH), _row_spec(f_in), _row_spec(1),
            _full_spec((1, f_in)), _full_spec((f_in, 1)),
        ],
        out_specs=_row_spec(1),
        out_shape=jax.ShapeDtypeStruct((N_NODES, 1), jnp.float32),
    )(aggl, aggr, p, dinv, b2.reshape(1, f_in), Wfc)


def _tc_pool(z, batch_i32, bfc):
    zp = jnp.concatenate([z[:, 0], jnp.zeros((240,), jnp.float32)]).reshape(80, 128)
    bp = jnp.concatenate(
        [batch_i32, jnp.full((240,), 1 << 20, jnp.int32)]).reshape(80, 128)
    return pl.pallas_call(
        _pool_body,
        in_specs=[
            pl.BlockSpec((80, 128), lambda: (0, 0)),
            pl.BlockSpec((80, 128), lambda: (0, 0)),
            pl.BlockSpec((1, 1), lambda: (0, 0)),
        ],
        out_specs=pl.BlockSpec((N_GRAPHS, 1), lambda: (0, 0)),
        out_shape=jax.ShapeDtypeStruct((N_GRAPHS, 1), jnp.float32),
    )(zp, bp, bfc.reshape(1, 1))


def kernel(x, edge_index, batch, W1, b1, W2, b2, Wfc, bfc):
    src = edge_index[0].astype(jnp.int32)
    dst = edge_index[1].astype(jnp.int32)
    batch_i32 = batch.astype(jnp.int32)

    n_pad = E_PAD - src.shape[0]
    src_pad = jnp.concatenate([src, jnp.full((n_pad,), N_NODES, jnp.int32)])
    dst_pad = jnp.concatenate([dst, jnp.zeros((n_pad,), jnp.int32)])
    srcl = (2 * src_pad).reshape(NCHUNK_ROWS, CHUNK)
    srcr = (2 * src_pad + 1).reshape(NCHUNK_ROWS, CHUNK)
    dstc = dst_pad.reshape(NCHUNK_ROWS, CHUNK)
    dst0 = jnp.where(dst_pad < HALF_R, dst_pad, JUNK).reshape(NCHUNK_ROWS, CHUNK)
    dst1 = jnp.where(dst_pad >= HALF_R, dst_pad - HALF_R,
                     JUNK).reshape(NCHUNK_ROWS, CHUNK)

    d0, d1 = _sc_degrees(dst0, dst1)
    corr = jnp.zeros((N_NODES, 1), jnp.float32).at[0, 0].set(float(n_pad))
    dinv, p1 = _tc_pre(d0[:N_NODES, :1], d1[:N_NODES, :1], corr, x, W1)

    a1l, a1r = _sc_aggregate(p1, srcl, srcr, dst0, dst1)
    p2 = _tc_mid(a1l[:N_NODES], a1r[:N_NODES], p1, dinv, b1, W2)

    a2l, a2r = _sc_aggregate(p2, srcl, srcr, dst0, dst1)
    z = _tc_fin(a2l[:N_NODES], a2r[:N_NODES], p2, dinv, b2, Wfc)

    return _tc_pool(z, batch_i32, bfc)


# speedup vs baseline: 4.2888x; 1.0001x over previous
import functools

import jax
import jax.numpy as jnp
from jax import lax
from jax.experimental import pallas as pl
from jax.experimental.pallas import tpu as pltpu
from jax.experimental.pallas import tpu_sc as plsc

N_NODES = 10000
N_GRAPHS = 64
ROW_BLK = 1000
CH = 128

NSUB = 16
CHUNK = 128
CHUNKS_PER_SUB = 160
E_PAD = NSUB * CHUNKS_PER_SUB * CHUNK
NCHUNK_ROWS = E_PAD // CHUNK
HALF_R = 5120
ACC_R = 5248
JUNK = HALF_R
NR_OUT = 2 * HALF_R
ZERO_PER_SUB = HALF_R // NSUB
DEG_R = 10112
DEG_PER_SUB = DEG_R // NSUB

@functools.lru_cache(maxsize=None)
def _sc_mesh():
    return plsc.VectorSubcoreMesh(core_axis_name="c", subcore_axis_name="s")


def _deg_body(dst0, dst1, ones, zeros, out_l, out_r, idx_d0, idx_d1, ones_v,
              acc):
    cid = lax.axis_index("c")
    sid = lax.axis_index("s")
    half = CHUNKS_PER_SUB // 2
    base = cid * (NCHUNK_ROWS // 2) + sid * half
    pltpu.sync_copy(dst0.at[pl.ds(base, half)], idx_d0)
    pltpu.sync_copy(dst1.at[pl.ds(base, half)], idx_d1)
    pltpu.sync_copy(ones, ones_v)

    for p, idx_d in ((0, idx_d0), (1, idx_d1)):
        pltpu.sync_copy(zeros.at[pl.ds(sid * ZERO_PER_SUB, ZERO_PER_SUB)],
                        acc.at[pl.ds(sid * ZERO_PER_SUB, ZERO_PER_SUB)])
        plsc.subcore_barrier()

        def step(j, carry):
            pltpu.sync_copy(ones_v, acc.at[idx_d.at[j]], add=True)
            return carry

        lax.fori_loop(0, half, step, 0)
        plsc.subcore_barrier()

        obase = p * HALF_R + sid * ZERO_PER_SUB

        @pl.when(cid == 0)
        def _():
            pltpu.sync_copy(acc.at[pl.ds(sid * ZERO_PER_SUB, ZERO_PER_SUB)],
                            out_l.at[pl.ds(obase, ZERO_PER_SUB)])

        @pl.when(cid == 1)
        def _():
            pltpu.sync_copy(acc.at[pl.ds(sid * ZERO_PER_SUB, ZERO_PER_SUB)],
                            out_r.at[pl.ds(obase, ZERO_PER_SUB)])
        plsc.subcore_barrier()


def _sc_degrees(dst0, dst1):
    deg_kernel = pl.kernel(
        _deg_body,
        out_type=[
            jax.ShapeDtypeStruct((NR_OUT, CH), jnp.float32),
            jax.ShapeDtypeStruct((NR_OUT, CH), jnp.float32),
        ],
        mesh=_sc_mesh(),
        scratch_types=[
            pltpu.VMEM((CHUNKS_PER_SUB // 2, CHUNK), jnp.int32),
            pltpu.VMEM((CHUNKS_PER_SUB // 2, CHUNK), jnp.int32),
            pltpu.VMEM((CHUNK, CH), jnp.float32),
            pltpu.VMEM_SHARED((ACC_R, CH), jnp.float32),
        ],
    )
    ones = jnp.ones((CHUNK, CH), jnp.float32)
    zeros = jnp.zeros((HALF_R, CH), jnp.float32)
    return deg_kernel(dst0, dst1, ones, zeros)


NBUF = 2


def _agg_body(table, srcl, srcr, dst0, dst1, zeros, out_l, out_r,
              idx_s, idx_d, b0, b1, acc, g0, g1, s0, s1):
    cid = lax.axis_index("c")
    sid = lax.axis_index("s")
    base = sid * CHUNKS_PER_SUB
    bufs = (b0, b1)
    gsem = (g0, g1)
    ssem = (s0, s1)
    ngroups = CHUNKS_PER_SUB // NBUF

    @pl.when(cid == 0)
    def _():
        pltpu.sync_copy(srcl.at[pl.ds(base, CHUNKS_PER_SUB)], idx_s)

    @pl.when(cid == 1)
    def _():
        pltpu.sync_copy(srcr.at[pl.ds(base, CHUNKS_PER_SUB)], idx_s)

    for p, dstp in ((0, dst0), (1, dst1)):
        pltpu.sync_copy(dstp.at[pl.ds(base, CHUNKS_PER_SUB)], idx_d)
        pltpu.sync_copy(zeros.at[pl.ds(sid * ZERO_PER_SUB, ZERO_PER_SUB)],
                        acc.at[pl.ds(sid * ZERO_PER_SUB, ZERO_PER_SUB)])
        plsc.subcore_barrier()

        for b in range(NBUF):
            pltpu.async_copy(table.at[idx_s.at[b]], bufs[b], gsem[b])

        def group(g, carry):
            jb = g * NBUF
            for b in range(NBUF):
                j = jb + b
                pltpu.make_async_copy(table.at[idx_s.at[j]], bufs[b],
                                      gsem[b]).wait()
                pltpu.async_copy(bufs[b], acc.at[idx_d.at[j]], ssem[b],
                                 add=True)
                pltpu.make_async_copy(bufs[b], acc.at[idx_d.at[j]],
                                      ssem[b]).wait()
                pltpu.async_copy(table.at[idx_s.at[j + NBUF]], bufs[b],
                                 gsem[b])
            return carry

        lax.fori_loop(0, ngroups - 1, group, 0)

        for b in range(NBUF):
            j = CHUNKS_PER_SUB - NBUF + b
            pltpu.make_async_copy(table.at[idx_s.at[j]], bufs[b],
                                  gsem[b]).wait()
            pltpu.sync_copy(bufs[b], acc.at[idx_d.at[j]], add=True)

        plsc.subcore_barrier()
        obase = p * HALF_R + sid * ZERO_PER_SUB

        @pl.when(cid == 0)
        def _():
            pltpu.sync_copy(acc.at[pl.ds(sid * ZERO_PER_SUB, ZERO_PER_SUB)],
                            out_l.at[pl.ds(obase, ZERO_PER_SUB)])

        @pl.when(cid == 1)
        def _():
            pltpu.sync_copy(acc.at[pl.ds(sid * ZERO_PER_SUB, ZERO_PER_SUB)],
                            out_r.at[pl.ds(obase, ZERO_PER_SUB)])
        plsc.subcore_barrier()


@functools.lru_cache(maxsize=None)
def _make_agg_kernel():
    return pl.kernel(
        _agg_body,
        out_type=[
            jax.ShapeDtypeStruct((NR_OUT, CH), jnp.float32),
            jax.ShapeDtypeStruct((NR_OUT, CH), jnp.float32),
        ],
        mesh=_sc_mesh(),
        scratch_types=[
            pltpu.VMEM((CHUNKS_PER_SUB, CHUNK), jnp.int32),
            pltpu.VMEM((CHUNKS_PER_SUB, CHUNK), jnp.int32),
            pltpu.VMEM((CHUNK, CH), jnp.float32),
            pltpu.VMEM((CHUNK, CH), jnp.float32),
            pltpu.VMEM_SHARED((ACC_R, CH), jnp.float32),
            pltpu.SemaphoreType.DMA,
            pltpu.SemaphoreType.DMA,
            pltpu.SemaphoreType.DMA,
            pltpu.SemaphoreType.DMA,
        ],
    )


def _sc_aggregate(rows_mat, srcl, srcr, dst0, dst1):
    table = jnp.concatenate(
        [rows_mat, jnp.zeros((1, 2 * CH), jnp.float32)]).reshape(-1, CH)
    zeros = jnp.zeros((HALF_R, CH), jnp.float32)
    return _make_agg_kernel()(table, srcl, srcr, dst0, dst1, zeros)


def _pre_body(d0_ref, d1_ref, corr_ref, x_ref, w_ref, dinv_ref, p_ref):
    deg = d0_ref[...] + d1_ref[...] + 1.0 - corr_ref[...]
    dinv = lax.rsqrt(deg)
    dinv_ref[...] = dinv
    p_ref[...] = jnp.dot(dinv * x_ref[...], w_ref[...],
                         preferred_element_type=jnp.float32)


def _mid_body(aggl_ref, aggr_ref, p_ref, dinv_ref, b_ref, w_ref, p2_ref):
    agg = jnp.concatenate([aggl_ref[...], aggr_ref[...]], axis=1)
    h = dinv_ref[...] * (agg + p_ref[...]) + b_ref[...]
    hs = dinv_ref[...] * jnp.maximum(h, 0.0)
    p2_ref[...] = jnp.dot(hs, w_ref[...], preferred_element_type=jnp.float32)


def _fin_body(aggl_ref, aggr_ref, p_ref, dinv_ref, b_ref, wfc_ref, z_ref):
    agg = jnp.concatenate([aggl_ref[...], aggr_ref[...]], axis=1)
    h = dinv_ref[...] * (agg + p_ref[...]) + b_ref[...]
    h = jnp.maximum(h, 0.0)
    z_ref[...] = jnp.dot(h, wfc_ref[...], preferred_element_type=jnp.float32)


def _pool_body(z_ref, bt_ref, bfc_ref, o_ref):
    z = z_ref[...]
    bt = bt_ref[...]
    gids = lax.broadcasted_iota(jnp.int32, (N_GRAPHS, z.shape[0], z.shape[1]), 0)
    m = bt[None, :, :] == gids
    s1 = jnp.sum(jnp.where(m, z[None, :, :], 0.0), axis=1)
    sums = jnp.sum(s1, axis=1, keepdims=True)
    c1 = jnp.sum(jnp.where(m, 1.0, 0.0), axis=1)
    cnts = jnp.sum(c1, axis=1, keepdims=True)
    o_ref[...] = sums / jnp.maximum(cnts, 1.0) + bfc_ref[...]


def _row_spec(width):
    return pl.BlockSpec((ROW_BLK, width), lambda i: (i, 0))


def _full_spec(shape):
    return pl.BlockSpec(shape, lambda i: tuple(0 for _ in shape))


def _tc_pre(d0, d1, corr, x, W1):
    f_in, f_out = W1.shape
    return pl.pallas_call(
        _pre_body,
        grid=(N_NODES // ROW_BLK,),
        in_specs=[_row_spec(1), _row_spec(1), _row_spec(1), _row_spec(f_in),
                  _full_spec((f_in, f_out))],
        out_specs=[_row_spec(1), _row_spec(f_out)],
        out_shape=[
            jax.ShapeDtypeStruct((N_NODES, 1), jnp.float32),
            jax.ShapeDtypeStruct((N_NODES, f_out), jnp.float32),
        ],
    )(d0, d1, corr, x, W1)


def _tc_mid(aggl, aggr, p, dinv, b1, W2):
    f_in, f_out = W2.shape
    return pl.pallas_call(
        _mid_body,
        grid=(N_NODES // ROW_BLK,),
        in_specs=[
            _row_spec(CH), _row_spec(CH), _row_spec(f_in), _row_spec(1),
            _full_spec((1, f_in)), _full_spec((f_in, f_out)),
        ],
        out_specs=_row_spec(f_out),
        out_shape=jax.ShapeDtypeStruct((N_NODES, f_out), jnp.float32),
    )(aggl, aggr, p, dinv, b1.reshape(1, f_in), W2)


def _tc_fin(aggl, aggr, p, dinv, b2, Wfc):
    f_in = Wfc.shape[0]
    return pl.pallas_call(
        _fin_body,
        grid=(N_NODES // ROW_BLK,),
        in_specs=[
            _row_spec(CH), _row_spec(CH), _row_spec(f_in), _row_spec(1),
            _full_spec((1, f_in)), _full_spec((f_in, 1)),
        ],
        out_specs=_row_spec(1),
        out_shape=jax.ShapeDtypeStruct((N_NODES, 1), jnp.float32),
    )(aggl, aggr, p, dinv, b2.reshape(1, f_in), Wfc)


def _tc_pool(z, batch_i32, bfc):
    zp = jnp.concatenate([z[:, 0], jnp.zeros((240,), jnp.float32)]).reshape(80, 128)
    bp = jnp.concatenate(
        [batch_i32, jnp.full((240,), 1 << 20, jnp.int32)]).reshape(80, 128)
    return pl.pallas_call(
        _pool_body,
        in_specs=[
            pl.BlockSpec((80, 128), lambda: (0, 0)),
            pl.BlockSpec((80, 128), lambda: (0, 0)),
            pl.BlockSpec((1, 1), lambda: (0, 0)),
        ],
        out_specs=pl.BlockSpec((N_GRAPHS, 1), lambda: (0, 0)),
        out_shape=jax.ShapeDtypeStruct((N_GRAPHS, 1), jnp.float32),
    )(zp, bp, bfc.reshape(1, 1))


def kernel(x, edge_index, batch, W1, b1, W2, b2, Wfc, bfc):
    src = edge_index[0].astype(jnp.int32)
    dst = edge_index[1].astype(jnp.int32)
    batch_i32 = batch.astype(jnp.int32)

    n_pad = E_PAD - src.shape[0]
    src_pad = jnp.concatenate([src, jnp.full((n_pad,), N_NODES, jnp.int32)])
    dst_pad = jnp.concatenate([dst, jnp.zeros((n_pad,), jnp.int32)])
    srcl = (2 * src_pad).reshape(NCHUNK_ROWS, CHUNK)
    srcr = (2 * src_pad + 1).reshape(NCHUNK_ROWS, CHUNK)
    dstc = dst_pad.reshape(NCHUNK_ROWS, CHUNK)
    dst0 = jnp.where(dst_pad < HALF_R, dst_pad, JUNK).reshape(NCHUNK_ROWS, CHUNK)
    dst1 = jnp.where(dst_pad >= HALF_R, dst_pad - HALF_R,
                     JUNK).reshape(NCHUNK_ROWS, CHUNK)

    d0, d1 = _sc_degrees(dst0, dst1)
    corr = jnp.zeros((N_NODES, 1), jnp.float32).at[0, 0].set(float(n_pad))
    dinv, p1 = _tc_pre(d0[:N_NODES, :1], d1[:N_NODES, :1], corr, x, W1)

    a1l, a1r = _sc_aggregate(p1, srcl, srcr, dst0, dst1)
    p2 = _tc_mid(a1l[:N_NODES], a1r[:N_NODES], p1, dinv, b1, W2)

    a2l, a2r = _sc_aggregate(p2, srcl, srcr, dst0, dst1)
    z = _tc_fin(a2l[:N_NODES], a2r[:N_NODES], p2, dinv, b2, Wfc)

    return _tc_pool(z, batch_i32, bfc)
